# baseline (speedup 1.0000x reference)
import jax
import jax.numpy as jnp
from jax.experimental import pallas as pl
from jax.experimental.pallas import tpu as pltpu

_N = 16
_B = 8
_D = 4


def _dma_pipeline(x_ref, o_ref, buf, sin, sout):
    rows = x_ref.shape[0] // _N

    def cp_in(i):
        return pltpu.make_async_copy(
            x_ref.at[pl.ds(i * rows, rows)], buf.at[i % _B], sin.at[i])

    def cp_out(i):
        return pltpu.make_async_copy(
            buf.at[i % _B], o_ref.at[pl.ds(i * rows, rows)], sout.at[i])

    for j in range(_D):
        cp_in(j).start()
    for i in range(_N):
        cp_in(i).wait()
        cp_out(i).start()
        j = i + _D
        if j < _N:
            if j - _B >= 0:
                cp_out(j - _B).wait()
            cp_in(j).start()
    for i in range(_N - _B, _N):
        cp_out(i).wait()


def kernel(x):
    b, c, f = x.shape
    return pl.pallas_call(
        _dma_pipeline,
        in_specs=[pl.BlockSpec(memory_space=pl.ANY)],
        out_specs=pl.BlockSpec(memory_space=pl.ANY),
        out_shape=jax.ShapeDtypeStruct((b, c, f), x.dtype),
        scratch_shapes=[
            pltpu.VMEM((_B, b // _N, c, f), x.dtype),
            pltpu.SemaphoreType.DMA((_N,)),
            pltpu.SemaphoreType.DMA((_N,)),
        ],
    )(x)
